# baseline (speedup 1.0000x reference)
import functools

import jax
import jax.numpy as jnp
from jax import lax
from jax.experimental import pallas as pl
from jax.experimental.pallas import tpu as pltpu
from jax.experimental.pallas import tpu_sc as plsc

F = 1024
D = 1024
K = 8
EPS = 1e-3
BB = 512
NH = 1
HB = BB // NH

NW = 32
CH = 4


def _dot_t(a, b):
    return jax.lax.dot_general(
        a, b, (((1,), (1,)), ((), ())), preferred_element_type=jnp.float32)


def _pursuit_body(x_ref, xs_ref, wout_ref, iout_ref):
    xs = xs_ref[...]
    xsb = xs.astype(jnp.bfloat16)
    col = jax.lax.broadcasted_iota(jnp.int32, (HB, F), 1)
    xh = [x_ref[pl.ds(h * HB, HB), :] for h in range(NH)]

    def iteration(w, x):
        residual = x - jnp.dot(w.astype(jnp.bfloat16), xsb,
                               preferred_element_type=jnp.float32)
        ip = _dot_t(residual.astype(jnp.bfloat16), xsb)
        idx = jnp.argmax(ip, axis=1)[:, None]
        mask = (w != 0.0) | (col == idx)
        grad = jnp.where(mask, ip, 0.0)
        c = jnp.dot(grad.astype(jnp.bfloat16), xsb,
                    preferred_element_type=jnp.float32)
        num = jnp.sum(c * residual, axis=1, keepdims=True)
        den = jnp.sum(c * c, axis=1, keepdims=True)
        step = num / jnp.maximum(den, EPS)
        return jax.nn.relu(w + step * grad)

    def body(t, ws):
        return tuple(iteration(w, x) for w, x in zip(ws, xh))

    ws = jax.lax.fori_loop(
        0, K, body, tuple(jnp.zeros((HB, F), jnp.float32) for _ in range(NH)))

    for h in range(NH):
        w = ws[h]
        rows = pl.ds(h * HB, HB)
        vals, inds = [], []
        for _ in range(K):
            mx = jnp.max(w, axis=1, keepdims=True)
            idx = jnp.min(jnp.where(w == mx, col, F), axis=1, keepdims=True)
            vals.append(mx)
            inds.append(idx)
            w = jnp.where(col == idx, -1.0, w)
        wout_ref[rows, :] = jnp.concatenate(vals, axis=1)
        iout_ref[rows, :] = jnp.concatenate(inds, axis=1)


def _make_decode(B):
    RPW = B // NW
    NCH = RPW // CH
    mesh = plsc.VectorSubcoreMesh(core_axis_name="c", subcore_axis_name="s")

    @functools.partial(
        pl.kernel, mesh=mesh,
        out_type=[
            jax.ShapeDtypeStruct((B, D), jnp.float32),
            jax.ShapeDtypeStruct((B, D), jnp.float32),
            jax.ShapeDtypeStruct((B,), jnp.float32),
        ],
        scratch_types=[
            pltpu.VMEM((RPW * K,), jnp.int32),
            pltpu.VMEM((RPW * K,), jnp.float32),
            pltpu.VMEM((CH * K, D), jnp.float32),
            pltpu.VMEM((CH * K, D), jnp.float32),
            pltpu.VMEM((CH, D), jnp.float32),
            pltpu.VMEM((CH, D), jnp.float32),
            pltpu.VMEM((CH, D), jnp.float32),
            pltpu.VMEM((RPW,), jnp.float32),
            pltpu.SemaphoreType.DMA,
            pltpu.SemaphoreType.DMA,
        ],
    )
    def decode(idx_hbm, w_hbm, xs_hbm, ys_hbm, y_hbm,
               xrec_hbm, yrec_hbm, loss_hbm,
               idx_v, w_v, rx_v, ry_v, ybuf, xo, yo, lbuf, sem1, sem2):
        wid = lax.axis_index("s") * 2 + lax.axis_index("c")
        base = wid * RPW
        pltpu.sync_copy(idx_hbm.at[pl.ds(base * K, RPW * K)], idx_v)
        pltpu.sync_copy(w_hbm.at[pl.ds(base * K, RPW * K)], w_v)

        lane_ids = lax.iota(jnp.int32, 16)
        grp = 16 // CH

        def chunk_body(ci, lvec):
            row0 = base + ci * CH
            off = ci * CH * K
            cp1 = pltpu.async_copy(
                xs_hbm.at[idx_v.at[pl.ds(off, CH * K)]], rx_v, sem1)
            cp2 = pltpu.async_copy(
                ys_hbm.at[idx_v.at[pl.ds(off, CH * K)]], ry_v, sem2)
            pltpu.sync_copy(y_hbm.at[pl.ds(row0, CH)], ybuf)
            cp1.wait()
            cp2.wait()
            for p in range(CH // 2):
                wpair = w_v[pl.ds(off + p * 2 * K, 16)]
                for rr in range(2):
                    r = p * 2 + rr

                    def inner(c16, lacc):
                        sl = pl.ds(c16 * 16, 16)
                        xa = jnp.zeros((16,), jnp.float32)
                        ya = jnp.zeros((16,), jnp.float32)
                        for j in range(K):
                            wv = wpair[rr * K + j]
                            xa = xa + wv * rx_v[r * K + j, sl]
                            ya = ya + wv * ry_v[r * K + j, sl]
                        xo[r, sl] = xa
                        yo[r, sl] = ya
                        d = ya - ybuf[r, sl]
                        return lacc + d * d

                    lacc = lax.fori_loop(0, D // 16, inner,
                                         jnp.zeros((16,), jnp.float32))
                    for sh in (8, 4, 2, 1):
                        lacc = lacc + lacc.at[(lane_ids + sh) & 15].get(
                            mode="promise_in_bounds")
                    lane = (ci * CH + r) % 16
                    lvec = jnp.where(lane_ids == lane, lacc, lvec)
            pltpu.sync_copy(xo, xrec_hbm.at[pl.ds(row0, CH)])
            pltpu.sync_copy(yo, yrec_hbm.at[pl.ds(row0, CH)])

            @pl.when(ci % grp == grp - 1)
            def _():
                lbuf[pl.ds((ci // grp) * 16, 16)] = lvec

            return lvec

        lax.fori_loop(0, NCH, chunk_body, jnp.zeros((16,), jnp.float32))
        pltpu.sync_copy(lbuf, loss_hbm.at[pl.ds(base, RPW)])

    return decode


@jax.jit
def kernel(x, y, xs, ys):
    B = x.shape[0]
    nblk = B // BB
    row_blk = lambda i: (i, 0)
    fixed = lambda i: (0, 0)
    weights, indices = pl.pallas_call(
        _pursuit_body,
        grid=(nblk,),
        in_specs=[
            pl.BlockSpec((BB, D), row_blk),
            pl.BlockSpec((F, D), fixed),
        ],
        out_specs=[
            pl.BlockSpec((BB, K), row_blk),
            pl.BlockSpec((BB, K), row_blk),
        ],
        out_shape=[
            jax.ShapeDtypeStruct((B, K), jnp.float32),
            jax.ShapeDtypeStruct((B, K), jnp.int32),
        ],
    )(x, xs)
    x_rec, y_rec, losses = _make_decode(B)(
        indices.reshape(B * K), weights.reshape(B * K), xs, ys, y)
    return weights, indices, x_rec, y_rec, losses


# speedup vs baseline: 1.9414x; 1.9414x over previous
import functools

import jax
import jax.numpy as jnp
from jax.experimental import pallas as pl
from jax.experimental.pallas import tpu as pltpu

F = 1024
D = 1024
K = 8
EPS = 1e-3
BB = 512
NH = 1
HB = BB // NH


def _dot_t(a, b):
    return jax.lax.dot_general(
        a, b, (((1,), (1,)), ((), ())), preferred_element_type=jnp.float32)


def _pursuit_body(x_ref, y_ref, xs_ref, ys_ref,
                  wout_ref, iout_ref, xrec_ref, yrec_ref, loss_ref):
    xs = xs_ref[...]
    ys = ys_ref[...]
    xsb = xs.astype(jnp.bfloat16)
    col = jax.lax.broadcasted_iota(jnp.int32, (HB, F), 1)
    xh = [x_ref[pl.ds(h * HB, HB), :] for h in range(NH)]

    def iteration(w, x):
        residual = x - jnp.dot(w.astype(jnp.bfloat16), xsb,
                               preferred_element_type=jnp.float32)
        ip = _dot_t(residual.astype(jnp.bfloat16), xsb)
        idx = jnp.argmax(ip, axis=1)[:, None]
        mask = (w != 0.0) | (col == idx)
        grad = jnp.where(mask, ip, 0.0)
        c = jnp.dot(grad.astype(jnp.bfloat16), xsb,
                    preferred_element_type=jnp.float32)
        num = jnp.sum(c * residual, axis=1, keepdims=True)
        den = jnp.sum(c * c, axis=1, keepdims=True)
        step = num / jnp.maximum(den, EPS)
        return jax.nn.relu(w + step * grad)

    def body(t, ws):
        return tuple(iteration(w, x) for w, x in zip(ws, xh))

    ws = tuple(jnp.zeros((HB, F), jnp.float32) for _ in range(NH))
    for t in range(K):
        ws = body(t, ws)

    for h in range(NH):
        w = ws[h]
        rows = pl.ds(h * HB, HB)
        xrec_ref[rows, :] = jnp.dot(w, xs, preferred_element_type=jnp.float32)
        yrec = jnp.dot(w, ys, preferred_element_type=jnp.float32)
        yrec_ref[rows, :] = yrec
        dy = yrec - y_ref[rows, :]
        loss_ref[rows, :] = jnp.sum(dy * dy, axis=1, keepdims=True)

        vals, inds = [], []
        for _ in range(K):
            mx = jnp.max(w, axis=1, keepdims=True)
            idx = jnp.min(jnp.where(w == mx, col, F), axis=1, keepdims=True)
            vals.append(mx)
            inds.append(idx)
            w = jnp.where(col == idx, -1.0, w)
        wout_ref[rows, :] = jnp.concatenate(vals, axis=1)
        iout_ref[rows, :] = jnp.concatenate(inds, axis=1)


@jax.jit
def kernel(x, y, xs, ys):
    B = x.shape[0]
    nblk = B // BB
    row_blk = lambda i: (i, 0)
    fixed = lambda i: (0, 0)
    weights, indices, x_rec, y_rec, losses = pl.pallas_call(
        _pursuit_body,
        grid=(nblk,),
        in_specs=[
            pl.BlockSpec((BB, D), row_blk),
            pl.BlockSpec((BB, D), row_blk),
            pl.BlockSpec((F, D), fixed),
            pl.BlockSpec((F, D), fixed),
        ],
        out_specs=[
            pl.BlockSpec((BB, K), row_blk),
            pl.BlockSpec((BB, K), row_blk),
            pl.BlockSpec((BB, D), row_blk),
            pl.BlockSpec((BB, D), row_blk),
            pl.BlockSpec((BB, 1), row_blk),
        ],
        out_shape=[
            jax.ShapeDtypeStruct((B, K), jnp.float32),
            jax.ShapeDtypeStruct((B, K), jnp.int32),
            jax.ShapeDtypeStruct((B, D), jnp.float32),
            jax.ShapeDtypeStruct((B, D), jnp.float32),
            jax.ShapeDtypeStruct((B, 1), jnp.float32),
        ],
    )(x, y, xs, ys)
    return weights, indices, x_rec, y_rec, losses.reshape(B)
